# Initial kernel scaffold; baseline (speedup 1.0000x reference)
import jax
import jax.numpy as jnp
from jax.experimental import pallas as pl


def kernel(node_features, edge_index, weight, bias):
    raise NotImplementedError("write your pallas kernel here")


# speedup vs baseline: 6.4029x; 6.4029x over previous
import functools
import math

import jax
import jax.numpy as jnp
from jax import lax
from jax.experimental import pallas as pl
from jax.experimental.pallas import tpu as pltpu
from jax.experimental.pallas import tpu_sc as plsc

N_NODES = 10000
N_EDGES = 320000
D_FEAT = 128

NC = 2
NS = 16
NW = NC * NS

EPW = N_EDGES // NW
K = 80
NCHUNK = EPW // K
NPAD = 10240
RPT = NPAD // NS
DUMMY = N_NODES
LANES = 16


def _sc_aggregate(nf_hbm, row_hbm, col_hbm, acc_hbm, cnt_hbm,
                  row_v, col_v, radj_v, ones_v, rows_v, acc_sh, cnt_sh, sem):
    cid = lax.axis_index("c")
    sid = lax.axis_index("s")
    wid = sid * NC + cid

    zero16 = jnp.zeros((LANES,), jnp.float32)
    one16 = jnp.ones((LANES,), jnp.float32)
    for i in range(K // LANES):
        ones_v[pl.ds(i * LANES, LANES)] = one16

    def _zrow(r, carry):
        for j in range(D_FEAT // LANES):
            rows_v[r, pl.ds(j * LANES, LANES)] = zero16
        return carry
    lax.fori_loop(0, K, _zrow, 0)

    base_r = sid * RPT
    for k in range(RPT // K):
        pltpu.sync_copy(rows_v, acc_sh.at[pl.ds(base_r + k * K, K), :])
    for k in range(RPT // D_FEAT):
        pltpu.sync_copy(rows_v.at[0], cnt_sh.at[pl.ds(base_r + k * D_FEAT, D_FEAT)])

    plsc.subcore_barrier()

    ebase = wid * EPW

    def _chunk(c, carry):
        base = ebase + c * K
        pltpu.sync_copy(row_hbm.at[pl.ds(base, K)], row_v)
        pltpu.sync_copy(col_hbm.at[pl.ds(base, K)], col_v)
        for i in range(K // LANES):
            sl = pl.ds(i * LANES, LANES)
            r = row_v[sl]
            cc = col_v[sl]
            radj_v[sl] = jnp.where(r == cc, DUMMY, r)
        pltpu.async_copy(nf_hbm.at[col_v], rows_v, sem).wait()
        pltpu.sync_copy(rows_v, acc_sh.at[radj_v], add=True)
        pltpu.sync_copy(ones_v, cnt_sh.at[radj_v], add=True)
        return carry
    lax.fori_loop(0, NCHUNK, _chunk, 0)

    plsc.subcore_barrier()

    pltpu.sync_copy(acc_sh.at[pl.ds(base_r, RPT), :], acc_hbm.at[cid, pl.ds(base_r, RPT), :])
    pltpu.sync_copy(cnt_sh.at[pl.ds(base_r, RPT)], cnt_hbm.at[cid, pl.ds(base_r, RPT)])


@functools.cache
def _make_sc_call():
    return functools.partial(
        pl.kernel,
        mesh=plsc.VectorSubcoreMesh(core_axis_name="c", subcore_axis_name="s"),
        out_type=[
            jax.ShapeDtypeStruct((NC, NPAD, D_FEAT), jnp.float32),
            jax.ShapeDtypeStruct((NC, NPAD), jnp.float32),
        ],
        scratch_types=[
            pltpu.VMEM((K,), jnp.int32),
            pltpu.VMEM((K,), jnp.int32),
            pltpu.VMEM((K,), jnp.int32),
            pltpu.VMEM((K,), jnp.float32),
            pltpu.VMEM((K, D_FEAT), jnp.float32),
            pltpu.VMEM_SHARED((NPAD, D_FEAT), jnp.float32),
            pltpu.VMEM_SHARED((NPAD,), jnp.float32),
            pltpu.SemaphoreType.DMA,
        ],
    )(_sc_aggregate)


BLK = 1024


def _tc_body(acc_ref, inv_ref, nf_ref, wt_ref, wb_ref, b_ref, o_ref):
    s = (acc_ref[0] + acc_ref[1]) * inv_ref[...]
    y = (jnp.dot(s, wt_ref[...], preferred_element_type=jnp.float32)
         + jnp.dot(nf_ref[...], wb_ref[...], preferred_element_type=jnp.float32)
         + b_ref[...])
    nrm = jnp.sqrt(jnp.sum(y * y, axis=1, keepdims=True))
    o_ref[...] = y / jnp.maximum(nrm, 1e-12)


def _tc_finish(acc, invb, nf_pad, w_top, w_bot, bias):
    grid = NPAD // BLK
    return pl.pallas_call(
        _tc_body,
        grid=(grid,),
        in_specs=[
            pl.BlockSpec((NC, BLK, D_FEAT), lambda i: (0, i, 0)),
            pl.BlockSpec((BLK, D_FEAT), lambda i: (i, 0)),
            pl.BlockSpec((BLK, D_FEAT), lambda i: (i, 0)),
            pl.BlockSpec((D_FEAT, D_FEAT), lambda i: (0, 0)),
            pl.BlockSpec((D_FEAT, D_FEAT), lambda i: (0, 0)),
            pl.BlockSpec((1, D_FEAT), lambda i: (0, 0)),
        ],
        out_specs=pl.BlockSpec((BLK, D_FEAT), lambda i: (i, 0)),
        out_shape=jax.ShapeDtypeStruct((NPAD, D_FEAT), jnp.float32),
    )(acc, invb, nf_pad, w_top, w_bot, bias)


def kernel(node_features, edge_index, weight, bias):
    row = edge_index[0]
    col = edge_index[1]
    acc, cnt = _make_sc_call()(node_features, row, col)
    inv = 1.0 / jnp.clip(cnt[0] + cnt[1], 1.0)
    invb = jnp.broadcast_to(inv[:, None], (NPAD, D_FEAT))
    nf_pad = jnp.pad(node_features, ((0, NPAD - N_NODES), (0, 0)))
    out = _tc_finish(acc, invb, nf_pad, weight[:D_FEAT], weight[D_FEAT:], bias)
    return out[:N_NODES]


# speedup vs baseline: 13.4059x; 2.0937x over previous
import functools
import math

import jax
import jax.numpy as jnp
from jax import lax
from jax.experimental import pallas as pl
from jax.experimental.pallas import tpu as pltpu
from jax.experimental.pallas import tpu_sc as plsc

N_NODES = 10000
N_EDGES = 320000
D_FEAT = 128

NC = 2
NS = 16
NW = NC * NS

EPW = N_EDGES // NW
K = 80
NCHUNK = EPW // K
NPAD = 10240
RPT = NPAD // NS
DUMMY = N_NODES
LANES = 16


def _sc_aggregate(nf_hbm, row_hbm, col_hbm, acc_hbm, cnt_hbm,
                  row_big, col_big, radj_a, radj_b, rows_a, rows_b,
                  ones_v, acc_sh, cnt_sh, sem_a, sem_b):
    cid = lax.axis_index("c")
    sid = lax.axis_index("s")
    wid = sid * NC + cid

    zero16 = jnp.zeros((LANES,), jnp.float32)
    ebase = wid * EPW
    pltpu.sync_copy(row_hbm.at[pl.ds(ebase, EPW)], row_big)
    pltpu.sync_copy(col_hbm.at[pl.ds(ebase, EPW)], col_big)

    def _zrow(r, carry):
        for j in range(D_FEAT // LANES):
            rows_a[r, pl.ds(j * LANES, LANES)] = zero16
        return carry
    lax.fori_loop(0, K, _zrow, 0)

    one16 = jnp.ones((LANES,), jnp.float32)
    for i in range(K // LANES):
        ones_v[pl.ds(i * LANES, LANES)] = one16

    base_r = sid * RPT
    for k in range(RPT // K):
        pltpu.sync_copy(rows_a, acc_sh.at[pl.ds(base_r + k * K, K), :])
    for k in range(RPT // D_FEAT):
        pltpu.sync_copy(rows_a.at[0], cnt_sh.at[pl.ds(base_r + k * D_FEAT, D_FEAT)])

    plsc.subcore_barrier()

    def _prep(c, radj_v, rows_v, gsem):
        base = c * K
        for i in range(K // LANES):
            sl = pl.ds(base + i * LANES, LANES)
            r = row_big[sl]
            cc = col_big[sl]
            radj_v[pl.ds(i * LANES, LANES)] = jnp.where(r == cc, DUMMY, r)
        pltpu.async_copy(nf_hbm.at[col_big.at[pl.ds(base, K)]], rows_v, gsem)

    def _finish(c, radj_v, rows_v, gsem):
        pltpu.make_async_copy(
            nf_hbm.at[col_big.at[pl.ds(c * K, K)]], rows_v, gsem).wait()
        pltpu.sync_copy(rows_v, acc_sh.at[radj_v], add=True)
        pltpu.sync_copy(ones_v, cnt_sh.at[radj_v], add=True)

    _prep(0, radj_a, rows_a, sem_a)

    def _pair(g, carry):
        c = 2 * g + 1
        _prep(c, radj_b, rows_b, sem_b)
        _finish(c - 1, radj_a, rows_a, sem_a)
        _prep(c + 1, radj_a, rows_a, sem_a)
        _finish(c, radj_b, rows_b, sem_b)
        return carry
    lax.fori_loop(0, (NCHUNK - 1) // 2, _pair, 0)
    _finish(NCHUNK - 1, radj_a, rows_a, sem_a)

    plsc.subcore_barrier()

    pltpu.sync_copy(acc_sh.at[pl.ds(base_r, RPT), :], acc_hbm.at[cid, pl.ds(base_r, RPT), :])
    pltpu.sync_copy(cnt_sh.at[pl.ds(base_r, RPT)], cnt_hbm.at[cid, pl.ds(base_r, RPT)])


@functools.cache
def _make_sc_call():
    return functools.partial(
        pl.kernel,
        mesh=plsc.VectorSubcoreMesh(core_axis_name="c", subcore_axis_name="s"),
        out_type=[
            jax.ShapeDtypeStruct((NC, NPAD, D_FEAT), jnp.float32),
            jax.ShapeDtypeStruct((NC, NPAD), jnp.float32),
        ],
        scratch_types=[
            pltpu.VMEM((EPW,), jnp.int32),
            pltpu.VMEM((EPW,), jnp.int32),
            pltpu.VMEM((K,), jnp.int32),
            pltpu.VMEM((K,), jnp.int32),
            pltpu.VMEM((K, D_FEAT), jnp.float32),
            pltpu.VMEM((K, D_FEAT), jnp.float32),
            pltpu.VMEM((K,), jnp.float32),
            pltpu.VMEM_SHARED((NPAD, D_FEAT), jnp.float32),
            pltpu.VMEM_SHARED((NPAD,), jnp.float32),
            pltpu.SemaphoreType.DMA,
            pltpu.SemaphoreType.DMA,
        ],
    )(_sc_aggregate)


BLK = 1024


def _tc_body(acc_ref, inv_ref, nf_ref, wt_ref, wb_ref, b_ref, o_ref):
    s = (acc_ref[0] + acc_ref[1]) * inv_ref[...]
    y = (jnp.dot(s, wt_ref[...], preferred_element_type=jnp.float32)
         + jnp.dot(nf_ref[...], wb_ref[...], preferred_element_type=jnp.float32)
         + b_ref[...])
    nrm = jnp.sqrt(jnp.sum(y * y, axis=1, keepdims=True))
    o_ref[...] = y / jnp.maximum(nrm, 1e-12)


def _tc_finish(acc, invb, nf_pad, w_top, w_bot, bias):
    grid = NPAD // BLK
    return pl.pallas_call(
        _tc_body,
        grid=(grid,),
        in_specs=[
            pl.BlockSpec((NC, BLK, D_FEAT), lambda i: (0, i, 0)),
            pl.BlockSpec((BLK, D_FEAT), lambda i: (i, 0)),
            pl.BlockSpec((BLK, D_FEAT), lambda i: (i, 0)),
            pl.BlockSpec((D_FEAT, D_FEAT), lambda i: (0, 0)),
            pl.BlockSpec((D_FEAT, D_FEAT), lambda i: (0, 0)),
            pl.BlockSpec((1, D_FEAT), lambda i: (0, 0)),
        ],
        out_specs=pl.BlockSpec((BLK, D_FEAT), lambda i: (i, 0)),
        out_shape=jax.ShapeDtypeStruct((NPAD, D_FEAT), jnp.float32),
    )(acc, invb, nf_pad, w_top, w_bot, bias)


def kernel(node_features, edge_index, weight, bias):
    row = edge_index[0]
    col = edge_index[1]
    acc, cnt = _make_sc_call()(node_features, row, col)
    inv = 1.0 / jnp.clip(cnt[0] + cnt[1], 1.0)
    invb = jnp.broadcast_to(inv[:, None], (NPAD, D_FEAT))
    nf_pad = jnp.pad(node_features, ((0, NPAD - N_NODES), (0, 0)))
    out = _tc_finish(acc, invb, nf_pad, weight[:D_FEAT], weight[D_FEAT:], bias)
    return out[:N_NODES]


# speedup vs baseline: 13.8452x; 1.0328x over previous
import functools
import math

import jax
import jax.numpy as jnp
from jax import lax
from jax.experimental import pallas as pl
from jax.experimental.pallas import tpu as pltpu
from jax.experimental.pallas import tpu_sc as plsc

N_NODES = 10000
N_EDGES = 320000
D_FEAT = 128

NC = 2
NS = 16
NW = NC * NS

EPW = N_EDGES // NW
K = 80
NCHUNK = EPW // K
NPAD = 10240
RPT = NPAD // NS
DUMMY = N_NODES
LANES = 16


def _sc_aggregate(nf_hbm, row_hbm, col_hbm, acc_hbm, cnt_hbm,
                  row_big, col_big, radj_a, radj_b, rows_a, rows_b,
                  ones_v, acc_sh, cnt_sh, sem_a, sem_b):
    cid = lax.axis_index("c")
    sid = lax.axis_index("s")
    wid = sid * NC + cid

    zero16 = jnp.zeros((LANES,), jnp.float32)
    ebase = wid * EPW
    pltpu.sync_copy(row_hbm.at[pl.ds(ebase, EPW)], row_big)
    pltpu.sync_copy(col_hbm.at[pl.ds(ebase, EPW)], col_big)

    def _zrow(r, carry):
        for j in range(D_FEAT // LANES):
            rows_a[r, pl.ds(j * LANES, LANES)] = zero16
        return carry
    lax.fori_loop(0, K, _zrow, 0)

    one16 = jnp.ones((LANES,), jnp.float32)
    for i in range(K // LANES):
        ones_v[pl.ds(i * LANES, LANES)] = one16

    base_r = sid * RPT
    for k in range(RPT // K):
        pltpu.sync_copy(rows_a, acc_sh.at[pl.ds(base_r + k * K, K), :])
    for k in range(RPT // D_FEAT):
        pltpu.sync_copy(rows_a.at[0], cnt_sh.at[pl.ds(base_r + k * D_FEAT, D_FEAT)])

    plsc.subcore_barrier()

    def _prep(c, radj_v, rows_v, gsem):
        base = c * K
        for i in range(K // LANES):
            sl = pl.ds(base + i * LANES, LANES)
            r = row_big[sl]
            cc = col_big[sl]
            radj_v[pl.ds(i * LANES, LANES)] = jnp.where(r == cc, DUMMY, r)
        pltpu.async_copy(nf_hbm.at[col_big.at[pl.ds(base, K)]], rows_v, gsem)

    def _finish(c, radj_v, rows_v, gsem):
        pltpu.make_async_copy(
            nf_hbm.at[col_big.at[pl.ds(c * K, K)]], rows_v, gsem).wait()
        pltpu.sync_copy(rows_v, acc_sh.at[radj_v], add=True)
        pltpu.sync_copy(ones_v, cnt_sh.at[radj_v], add=True)

    _prep(0, radj_a, rows_a, sem_a)

    def _pair(g, carry):
        c = 2 * g + 1
        _prep(c, radj_b, rows_b, sem_b)
        _finish(c - 1, radj_a, rows_a, sem_a)
        _prep(c + 1, radj_a, rows_a, sem_a)
        _finish(c, radj_b, rows_b, sem_b)
        return carry
    lax.fori_loop(0, (NCHUNK - 1) // 2, _pair, 0)
    _finish(NCHUNK - 1, radj_a, rows_a, sem_a)

    plsc.subcore_barrier()

    pltpu.sync_copy(acc_sh.at[pl.ds(base_r, RPT), :], acc_hbm.at[cid, pl.ds(base_r, RPT), :])
    pltpu.sync_copy(cnt_sh.at[pl.ds(base_r, RPT)], cnt_hbm.at[cid, pl.ds(base_r, RPT)])


@functools.cache
def _make_sc_call():
    return functools.partial(
        pl.kernel,
        mesh=plsc.VectorSubcoreMesh(core_axis_name="c", subcore_axis_name="s"),
        out_type=[
            jax.ShapeDtypeStruct((NC, NPAD, D_FEAT), jnp.float32),
            jax.ShapeDtypeStruct((NC, NPAD), jnp.float32),
        ],
        scratch_types=[
            pltpu.VMEM((EPW,), jnp.int32),
            pltpu.VMEM((EPW,), jnp.int32),
            pltpu.VMEM((K,), jnp.int32),
            pltpu.VMEM((K,), jnp.int32),
            pltpu.VMEM((K, D_FEAT), jnp.float32),
            pltpu.VMEM((K, D_FEAT), jnp.float32),
            pltpu.VMEM((K,), jnp.float32),
            pltpu.VMEM_SHARED((NPAD, D_FEAT), jnp.float32),
            pltpu.VMEM_SHARED((NPAD,), jnp.float32),
            pltpu.SemaphoreType.DMA,
            pltpu.SemaphoreType.DMA,
        ],
    )(_sc_aggregate)


BLK = 1000


def _tc_body(acc_ref, inv_ref, nf_ref, wt_ref, wb_ref, b_ref, o_ref):
    s = (acc_ref[0] + acc_ref[1]) * inv_ref[...]
    y = (jnp.dot(s, wt_ref[...], preferred_element_type=jnp.float32)
         + jnp.dot(nf_ref[...], wb_ref[...], preferred_element_type=jnp.float32)
         + b_ref[...])
    nrm = jnp.sqrt(jnp.sum(y * y, axis=1, keepdims=True))
    o_ref[...] = y / jnp.maximum(nrm, 1e-12)


def _tc_finish(acc, invb, nf, w_top, w_bot, bias):
    grid = N_NODES // BLK
    return pl.pallas_call(
        _tc_body,
        grid=(grid,),
        in_specs=[
            pl.BlockSpec((NC, BLK, D_FEAT), lambda i: (0, i, 0)),
            pl.BlockSpec((BLK, D_FEAT), lambda i: (i, 0)),
            pl.BlockSpec((BLK, D_FEAT), lambda i: (i, 0)),
            pl.BlockSpec((D_FEAT, D_FEAT), lambda i: (0, 0)),
            pl.BlockSpec((D_FEAT, D_FEAT), lambda i: (0, 0)),
            pl.BlockSpec((1, D_FEAT), lambda i: (0, 0)),
        ],
        out_specs=pl.BlockSpec((BLK, D_FEAT), lambda i: (i, 0)),
        out_shape=jax.ShapeDtypeStruct((N_NODES, D_FEAT), jnp.float32),
    )(acc, invb, nf, w_top, w_bot, bias)


def kernel(node_features, edge_index, weight, bias):
    row = edge_index[0]
    col = edge_index[1]
    acc, cnt = _make_sc_call()(node_features, row, col)
    inv = 1.0 / jnp.clip(cnt[0, :N_NODES] + cnt[1, :N_NODES], 1.0)
    invb = jnp.broadcast_to(inv[:, None], (N_NODES, D_FEAT))
    return _tc_finish(acc, invb, node_features, weight[:D_FEAT], weight[D_FEAT:], bias)


# speedup vs baseline: 14.9913x; 1.0828x over previous
import functools
import math

import jax
import jax.numpy as jnp
from jax import lax
from jax.experimental import pallas as pl
from jax.experimental.pallas import tpu as pltpu
from jax.experimental.pallas import tpu_sc as plsc

N_NODES = 10000
N_EDGES = 320000
D_FEAT = 128

NC = 2
NS = 16
NW = NC * NS

EPW = N_EDGES // NW
K = 128
NCHUNK = EPW // K
KTAIL = EPW - NCHUNK * K
NPAD = 10240
RPT = NPAD // NS
DUMMY = N_NODES
LANES = 16


def _sc_aggregate(nf_hbm, packed_hbm, acc_hbm, cnt_hbm,
                  packed_big, radj_a, radj_b, radj_t, col_a, col_b, col_t,
                  rows_a, rows_b, ones_v, acc_sh, cnt_sh, sem_a, sem_b):
    cid = lax.axis_index("c")
    sid = lax.axis_index("s")
    wid = sid * NC + cid

    zero16 = jnp.zeros((LANES,), jnp.float32)
    ebase = wid * EPW
    pltpu.sync_copy(packed_hbm.at[pl.ds(ebase, EPW)], packed_big)

    def _zrow(r, carry):
        for j in range(D_FEAT // LANES):
            rows_a[r, pl.ds(j * LANES, LANES)] = zero16
        return carry
    lax.fori_loop(0, K, _zrow, 0)

    one16 = jnp.ones((LANES,), jnp.float32)
    for i in range(K // LANES):
        ones_v[pl.ds(i * LANES, LANES)] = one16

    base_r = sid * RPT
    for k in range(RPT // K):
        pltpu.sync_copy(rows_a, acc_sh.at[pl.ds(base_r + k * K, K), :])
    for k in range(RPT // D_FEAT):
        pltpu.sync_copy(rows_a.at[0], cnt_sh.at[pl.ds(base_r + k * D_FEAT, D_FEAT)])

    plsc.subcore_barrier()

    def _unpack(base, i, radj_v, col_v):
        sl = pl.ds(base + i * LANES, LANES)
        p = packed_big[sl]
        cc = lax.bitwise_and(p, 16383)
        r = lax.shift_right_logical(p, 14)
        dst = pl.ds(i * LANES, LANES)
        col_v[dst] = cc
        radj_v[dst] = jnp.where(r == cc, DUMMY, r)

    def _prep(c, radj_v, col_v, rows_v, gsem):
        base = c * K
        for i in range(K // LANES):
            _unpack(base, i, radj_v, col_v)
        pltpu.async_copy(nf_hbm.at[col_v], rows_v, gsem)

    def _finish(radj_v, col_v, rows_v, gsem):
        pltpu.make_async_copy(nf_hbm.at[col_v], rows_v, gsem).wait()
        pltpu.sync_copy(rows_v, acc_sh.at[radj_v], add=True)
        pltpu.sync_copy(ones_v, cnt_sh.at[radj_v], add=True)

    _prep(0, radj_a, col_a, rows_a, sem_a)

    def _pair(g, carry):
        c = 2 * g + 1
        _prep(c, radj_b, col_b, rows_b, sem_b)
        _finish(radj_a, col_a, rows_a, sem_a)
        _prep(c + 1, radj_a, col_a, rows_a, sem_a)
        _finish(radj_b, col_b, rows_b, sem_b)
        return carry
    lax.fori_loop(0, (NCHUNK - 2) // 2, _pair, 0)
    _prep(NCHUNK - 1, radj_b, col_b, rows_b, sem_b)
    _finish(radj_a, col_a, rows_a, sem_a)
    _finish(radj_b, col_b, rows_b, sem_b)

    tbase = NCHUNK * K
    for i in range(KTAIL // LANES):
        _unpack(tbase, i, radj_t, col_t)
    pltpu.async_copy(
        nf_hbm.at[col_t], rows_a.at[pl.ds(0, KTAIL), :], sem_a).wait()
    pltpu.sync_copy(rows_a.at[pl.ds(0, KTAIL), :], acc_sh.at[radj_t], add=True)
    pltpu.sync_copy(ones_v.at[pl.ds(0, KTAIL)], cnt_sh.at[radj_t], add=True)

    plsc.subcore_barrier()

    pltpu.sync_copy(acc_sh.at[pl.ds(base_r, RPT), :], acc_hbm.at[cid, pl.ds(base_r, RPT), :])
    pltpu.sync_copy(cnt_sh.at[pl.ds(base_r, RPT)], cnt_hbm.at[cid, pl.ds(base_r, RPT)])


@functools.cache
def _make_sc_call():
    return functools.partial(
        pl.kernel,
        mesh=plsc.VectorSubcoreMesh(core_axis_name="c", subcore_axis_name="s"),
        out_type=[
            jax.ShapeDtypeStruct((NC, NPAD, D_FEAT), jnp.float32),
            jax.ShapeDtypeStruct((NC, NPAD), jnp.float32),
        ],
        scratch_types=[
            pltpu.VMEM((EPW,), jnp.int32),
            pltpu.VMEM((K,), jnp.int32),
            pltpu.VMEM((K,), jnp.int32),
            pltpu.VMEM((KTAIL,), jnp.int32),
            pltpu.VMEM((K,), jnp.int32),
            pltpu.VMEM((K,), jnp.int32),
            pltpu.VMEM((KTAIL,), jnp.int32),
            pltpu.VMEM((K, D_FEAT), jnp.float32),
            pltpu.VMEM((K, D_FEAT), jnp.float32),
            pltpu.VMEM((K,), jnp.float32),
            pltpu.VMEM_SHARED((NPAD, D_FEAT), jnp.float32),
            pltpu.VMEM_SHARED((NPAD,), jnp.float32),
            pltpu.SemaphoreType.DMA,
            pltpu.SemaphoreType.DMA,
        ],
    )(_sc_aggregate)


BLK = 1000


def _tc_body(acc_ref, inv_ref, nf_ref, wt_ref, wb_ref, b_ref, o_ref):
    s = (acc_ref[0] + acc_ref[1]) * inv_ref[...]
    y = (jnp.dot(s, wt_ref[...], preferred_element_type=jnp.float32)
         + jnp.dot(nf_ref[...], wb_ref[...], preferred_element_type=jnp.float32)
         + b_ref[...])
    nrm = jnp.sqrt(jnp.sum(y * y, axis=1, keepdims=True))
    o_ref[...] = y / jnp.maximum(nrm, 1e-12)


def _tc_finish(acc, invb, nf, w_top, w_bot, bias):
    grid = N_NODES // BLK
    return pl.pallas_call(
        _tc_body,
        grid=(grid,),
        in_specs=[
            pl.BlockSpec((NC, BLK, D_FEAT), lambda i: (0, i, 0)),
            pl.BlockSpec((BLK, D_FEAT), lambda i: (i, 0)),
            pl.BlockSpec((BLK, D_FEAT), lambda i: (i, 0)),
            pl.BlockSpec((D_FEAT, D_FEAT), lambda i: (0, 0)),
            pl.BlockSpec((D_FEAT, D_FEAT), lambda i: (0, 0)),
            pl.BlockSpec((1, D_FEAT), lambda i: (0, 0)),
        ],
        out_specs=pl.BlockSpec((BLK, D_FEAT), lambda i: (i, 0)),
        out_shape=jax.ShapeDtypeStruct((N_NODES, D_FEAT), jnp.float32),
    )(acc, invb, nf, w_top, w_bot, bias)


def kernel(node_features, edge_index, weight, bias):
    row = edge_index[0]
    col = edge_index[1]
    packed = row * 16384 + col
    acc, cnt = _make_sc_call()(node_features, packed)
    inv = 1.0 / jnp.clip(cnt[0, :N_NODES] + cnt[1, :N_NODES], 1.0)
    invb = jnp.broadcast_to(inv[:, None], (N_NODES, D_FEAT))
    return _tc_finish(acc, invb, node_features, weight[:D_FEAT], weight[D_FEAT:], bias)


# speedup vs baseline: 15.5396x; 1.0366x over previous
import functools
import math

import jax
import jax.numpy as jnp
from jax import lax
from jax.experimental import pallas as pl
from jax.experimental.pallas import tpu as pltpu
from jax.experimental.pallas import tpu_sc as plsc

N_NODES = 10000
N_EDGES = 320000
D_FEAT = 128

NC = 2
NS = 16
NW = NC * NS

EPW = N_EDGES // NW
K = 128
NCHUNK = EPW // K
KTAIL = EPW - NCHUNK * K
NPAD = 10240
RPT = NPAD // NS
DUMMY = N_NODES
LANES = 16


def _sc_aggregate(nf_hbm, packed_hbm, acc_hbm, cnt_hbm,
                  packed_big, radj_a, radj_b, radj_t, col_a, col_b, col_t,
                  rows_a, rows_b, ones_v, acc_sh, cnt_sh, sem_a, sem_b):
    cid = lax.axis_index("c")
    sid = lax.axis_index("s")
    wid = sid * NC + cid

    zero16 = jnp.zeros((LANES,), jnp.float32)
    ebase = wid * EPW
    pltpu.sync_copy(packed_hbm.at[pl.ds(ebase, EPW)], packed_big)

    def _zrow(r, carry):
        for j in range(D_FEAT // LANES):
            rows_a[r, pl.ds(j * LANES, LANES)] = zero16
        return carry
    lax.fori_loop(0, K, _zrow, 0)

    one16 = jnp.ones((LANES,), jnp.float32)
    for i in range(K // LANES):
        ones_v[pl.ds(i * LANES, LANES)] = one16

    base_r = sid * RPT
    for k in range(RPT // K):
        pltpu.sync_copy(rows_a, acc_sh.at[pl.ds(base_r + k * K, K), :])
    for k in range(RPT // D_FEAT):
        pltpu.sync_copy(rows_a.at[0], cnt_sh.at[pl.ds(base_r + k * D_FEAT, D_FEAT)])

    plsc.subcore_barrier()

    def _unpack(base, i, radj_v, col_v):
        sl = pl.ds(base + i * LANES, LANES)
        p = packed_big[sl]
        cc = lax.bitwise_and(p, 16383)
        r = lax.shift_right_logical(p, 14)
        dst = pl.ds(i * LANES, LANES)
        col_v[dst] = cc
        radj_v[dst] = jnp.where(r == cc, DUMMY, r)

    def _prep(c, radj_v, col_v, rows_v, gsem):
        base = c * K
        for i in range(K // LANES):
            _unpack(base, i, radj_v, col_v)
        pltpu.async_copy(nf_hbm.at[col_v], rows_v, gsem)

    def _finish(radj_v, col_v, rows_v, gsem):
        pltpu.make_async_copy(nf_hbm.at[col_v], rows_v, gsem).wait()
        pltpu.sync_copy(rows_v, acc_sh.at[radj_v], add=True)
        pltpu.sync_copy(ones_v, cnt_sh.at[radj_v], add=True)

    _prep(0, radj_a, col_a, rows_a, sem_a)

    def _pair(g, carry):
        c = 2 * g + 1
        _prep(c, radj_b, col_b, rows_b, sem_b)
        _finish(radj_a, col_a, rows_a, sem_a)
        _prep(c + 1, radj_a, col_a, rows_a, sem_a)
        _finish(radj_b, col_b, rows_b, sem_b)
        return carry
    lax.fori_loop(0, (NCHUNK - 2) // 2, _pair, 0)
    _prep(NCHUNK - 1, radj_b, col_b, rows_b, sem_b)
    _finish(radj_a, col_a, rows_a, sem_a)
    _finish(radj_b, col_b, rows_b, sem_b)

    tbase = NCHUNK * K
    for i in range(KTAIL // LANES):
        _unpack(tbase, i, radj_t, col_t)
    pltpu.async_copy(
        nf_hbm.at[col_t], rows_a.at[pl.ds(0, KTAIL), :], sem_a).wait()
    pltpu.sync_copy(rows_a.at[pl.ds(0, KTAIL), :], acc_sh.at[radj_t], add=True)
    pltpu.sync_copy(ones_v.at[pl.ds(0, KTAIL)], cnt_sh.at[radj_t], add=True)

    plsc.subcore_barrier()

    pltpu.sync_copy(acc_sh.at[pl.ds(base_r, RPT), :], acc_hbm.at[cid, pl.ds(base_r, RPT), :])
    pltpu.sync_copy(cnt_sh.at[pl.ds(base_r, RPT)], cnt_hbm.at[cid, pl.ds(base_r, RPT)])


@functools.cache
def _make_sc_call():
    return functools.partial(
        pl.kernel,
        mesh=plsc.VectorSubcoreMesh(core_axis_name="c", subcore_axis_name="s"),
        out_type=[
            jax.ShapeDtypeStruct((NC, NPAD, D_FEAT), jnp.float32),
            jax.ShapeDtypeStruct((NC, NPAD), jnp.float32),
        ],
        scratch_types=[
            pltpu.VMEM((EPW,), jnp.int32),
            pltpu.VMEM((K,), jnp.int32),
            pltpu.VMEM((K,), jnp.int32),
            pltpu.VMEM((KTAIL,), jnp.int32),
            pltpu.VMEM((K,), jnp.int32),
            pltpu.VMEM((K,), jnp.int32),
            pltpu.VMEM((KTAIL,), jnp.int32),
            pltpu.VMEM((K, D_FEAT), jnp.float32),
            pltpu.VMEM((K, D_FEAT), jnp.float32),
            pltpu.VMEM((K,), jnp.float32),
            pltpu.VMEM_SHARED((NPAD, D_FEAT), jnp.float32),
            pltpu.VMEM_SHARED((NPAD,), jnp.float32),
            pltpu.SemaphoreType.DMA,
            pltpu.SemaphoreType.DMA,
        ],
    )(_sc_aggregate)


BLK = 1024


def _tc_body(acc_ref, cnt_ref, nf_ref, wt_ref, wb_ref, b_ref, o_ref):
    cb = cnt_ref[0] + cnt_ref[1]
    inv = 1.0 / jnp.maximum(cb, 1.0)
    invb = jnp.broadcast_to(inv[:, :, None], (BLK // D_FEAT, D_FEAT, D_FEAT))
    s = (acc_ref[0] + acc_ref[1]).reshape(BLK // D_FEAT, D_FEAT, D_FEAT) * invb
    s = s.reshape(BLK, D_FEAT)
    y = (jnp.dot(s, wt_ref[...], preferred_element_type=jnp.float32)
         + jnp.dot(nf_ref[...], wb_ref[...], preferred_element_type=jnp.float32)
         + b_ref[...])
    nrm = jnp.sqrt(jnp.sum(y * y, axis=1, keepdims=True))
    o_ref[...] = y / jnp.maximum(nrm, 1e-12)


def _tc_finish(acc, cnt2d, nf, w_top, w_bot, bias):
    grid = NPAD // BLK
    return pl.pallas_call(
        _tc_body,
        grid=(grid,),
        in_specs=[
            pl.BlockSpec((NC, BLK, D_FEAT), lambda i: (0, i, 0)),
            pl.BlockSpec((NC, BLK // D_FEAT, D_FEAT), lambda i: (0, i, 0)),
            pl.BlockSpec((BLK, D_FEAT), lambda i: (i, 0)),
            pl.BlockSpec((D_FEAT, D_FEAT), lambda i: (0, 0)),
            pl.BlockSpec((D_FEAT, D_FEAT), lambda i: (0, 0)),
            pl.BlockSpec((1, D_FEAT), lambda i: (0, 0)),
        ],
        out_specs=pl.BlockSpec((BLK, D_FEAT), lambda i: (i, 0)),
        out_shape=jax.ShapeDtypeStruct((N_NODES, D_FEAT), jnp.float32),
    )(acc, cnt2d, nf, w_top, w_bot, bias)


def kernel(node_features, edge_index, weight, bias):
    row = edge_index[0]
    col = edge_index[1]
    packed = row * 16384 + col
    acc, cnt = _make_sc_call()(node_features, packed)
    cnt2d = cnt.reshape(NC, NPAD // D_FEAT, D_FEAT)
    return _tc_finish(acc, cnt2d, node_features, weight[:D_FEAT], weight[D_FEAT:], bias)


# speedup vs baseline: 16.5580x; 1.0655x over previous
import functools
import math

import jax
import jax.numpy as jnp
from jax import lax
from jax.experimental import pallas as pl
from jax.experimental.pallas import tpu as pltpu
from jax.experimental.pallas import tpu_sc as plsc

N_NODES = 10000
N_EDGES = 320000
D_FEAT = 128

NC = 2
NS = 16
NW = NC * NS

EPW = N_EDGES // NW
K = 80
NCHUNK = EPW // K
NBUF = 3
NPAD = 10240
RPT = NPAD // NS
DUMMY = N_NODES
LANES = 16


def _sc_aggregate(nf_hbm, packed_hbm, acc_hbm, cnt_hbm,
                  packed_big, radj0, radj1, radj2, col0, col1, col2,
                  rows0, rows1, rows2, ones_v, acc_sh, cnt_sh,
                  gsem0, gsem1, gsem2, ssem0, ssem1, ssem2,
                  csem0, csem1, csem2):
    radj = [radj0, radj1, radj2]
    col = [col0, col1, col2]
    rows = [rows0, rows1, rows2]
    gsem = [gsem0, gsem1, gsem2]
    ssem = [ssem0, ssem1, ssem2]
    csem = [csem0, csem1, csem2]
    cid = lax.axis_index("c")
    sid = lax.axis_index("s")
    wid = sid * NC + cid

    zero16 = jnp.zeros((LANES,), jnp.float32)
    ebase = wid * EPW
    pltpu.sync_copy(packed_hbm.at[pl.ds(ebase, EPW)], packed_big)

    def _zrow(r, carry):
        for j in range(D_FEAT // LANES):
            rows[0][r, pl.ds(j * LANES, LANES)] = zero16
        return carry
    lax.fori_loop(0, K, _zrow, 0)

    one16 = jnp.ones((LANES,), jnp.float32)
    for i in range(K // LANES):
        ones_v[pl.ds(i * LANES, LANES)] = one16

    base_r = sid * RPT
    for k in range(RPT // K):
        pltpu.sync_copy(rows[0], acc_sh.at[pl.ds(base_r + k * K, K), :])
    for k in range(RPT // D_FEAT):
        pltpu.sync_copy(rows[0].at[0], cnt_sh.at[pl.ds(base_r + k * D_FEAT, D_FEAT)])

    plsc.subcore_barrier()

    def _unpack(base, i, radj_v, col_v):
        sl = pl.ds(base + i * LANES, LANES)
        p = packed_big[sl]
        cc = lax.bitwise_and(p, 16383)
        r = lax.shift_right_logical(p, 14)
        dst = pl.ds(i * LANES, LANES)
        col_v[dst] = cc
        radj_v[dst] = jnp.where(r == cc, DUMMY, r)

    def _prep(c, j):
        base = c * K
        for i in range(K // LANES):
            _unpack(base, i, radj[j], col[j])
        pltpu.async_copy(nf_hbm.at[col[j]], rows[j], gsem[j])

    def _wait_scat(j):
        pltpu.make_async_copy(rows[j], acc_sh.at[radj[j]], ssem[j]).wait()
        pltpu.make_async_copy(ones_v, cnt_sh.at[radj[j]], csem[j]).wait()

    def _scat(j):
        pltpu.make_async_copy(nf_hbm.at[col[j]], rows[j], gsem[j]).wait()
        pltpu.async_copy(rows[j], acc_sh.at[radj[j]], ssem[j], add=True)
        pltpu.async_copy(ones_v, cnt_sh.at[radj[j]], csem[j], add=True)

    def _body(c, j, first=False, last=False):
        jn = (j + 1) % NBUF
        if not first:
            _wait_scat(jn)
        if not last:
            _prep(c + 1, jn)
        _scat(j)

    _prep(0, 0)
    _body(0, 0, first=True)
    _body(1, 1, first=True)

    def _triple(g, carry):
        c = 3 * g + 2
        _body(c, 2)
        _body(c + 1, 0)
        _body(c + 2, 1)
        return carry
    lax.fori_loop(0, (NCHUNK - 5) // 3, _triple, 0)
    _body(NCHUNK - 3, 2)
    _body(NCHUNK - 2, 0)
    _body(NCHUNK - 1, 1, last=True)
    _wait_scat(0)
    _wait_scat(1)

    plsc.subcore_barrier()

    pltpu.sync_copy(acc_sh.at[pl.ds(base_r, RPT), :], acc_hbm.at[cid, pl.ds(base_r, RPT), :])
    pltpu.sync_copy(cnt_sh.at[pl.ds(base_r, RPT)], cnt_hbm.at[cid, pl.ds(base_r, RPT)])


@functools.cache
def _make_sc_call():
    return functools.partial(
        pl.kernel,
        mesh=plsc.VectorSubcoreMesh(core_axis_name="c", subcore_axis_name="s"),
        out_type=[
            jax.ShapeDtypeStruct((NC, NPAD, D_FEAT), jnp.float32),
            jax.ShapeDtypeStruct((NC, NPAD), jnp.float32),
        ],
        scratch_types=(
            [pltpu.VMEM((EPW,), jnp.int32)]
            + [pltpu.VMEM((K,), jnp.int32)] * NBUF
            + [pltpu.VMEM((K,), jnp.int32)] * NBUF
            + [pltpu.VMEM((K, D_FEAT), jnp.float32)] * NBUF
            + [pltpu.VMEM((K,), jnp.float32)]
            + [pltpu.VMEM_SHARED((NPAD, D_FEAT), jnp.float32)]
            + [pltpu.VMEM_SHARED((NPAD,), jnp.float32)]
            + [pltpu.SemaphoreType.DMA] * (3 * NBUF)
        ),
    )(_sc_aggregate)


BLK = 1024


def _tc_body(acc_ref, cnt_ref, nf_ref, wt_ref, wb_ref, b_ref, o_ref):
    cb = cnt_ref[0] + cnt_ref[1]
    inv = 1.0 / jnp.maximum(cb, 1.0)
    invb = jnp.broadcast_to(inv[:, :, None], (BLK // D_FEAT, D_FEAT, D_FEAT))
    s = (acc_ref[0] + acc_ref[1]).reshape(BLK // D_FEAT, D_FEAT, D_FEAT) * invb
    s = s.reshape(BLK, D_FEAT)
    y = (jnp.dot(s, wt_ref[...], preferred_element_type=jnp.float32)
         + jnp.dot(nf_ref[...], wb_ref[...], preferred_element_type=jnp.float32)
         + b_ref[...])
    nrm = jnp.sqrt(jnp.sum(y * y, axis=1, keepdims=True))
    o_ref[...] = y / jnp.maximum(nrm, 1e-12)


def _tc_finish(acc, cnt2d, nf, w_top, w_bot, bias):
    grid = NPAD // BLK
    return pl.pallas_call(
        _tc_body,
        grid=(grid,),
        in_specs=[
            pl.BlockSpec((NC, BLK, D_FEAT), lambda i: (0, i, 0)),
            pl.BlockSpec((NC, BLK // D_FEAT, D_FEAT), lambda i: (0, i, 0)),
            pl.BlockSpec((BLK, D_FEAT), lambda i: (i, 0)),
            pl.BlockSpec((D_FEAT, D_FEAT), lambda i: (0, 0)),
            pl.BlockSpec((D_FEAT, D_FEAT), lambda i: (0, 0)),
            pl.BlockSpec((1, D_FEAT), lambda i: (0, 0)),
        ],
        out_specs=pl.BlockSpec((BLK, D_FEAT), lambda i: (i, 0)),
        out_shape=jax.ShapeDtypeStruct((N_NODES, D_FEAT), jnp.float32),
    )(acc, cnt2d, nf, w_top, w_bot, bias)


def kernel(node_features, edge_index, weight, bias):
    row = edge_index[0]
    col = edge_index[1]
    packed = row * 16384 + col
    acc, cnt = _make_sc_call()(node_features, packed)
    cnt2d = cnt.reshape(NC, NPAD // D_FEAT, D_FEAT)
    return _tc_finish(acc, cnt2d, node_features, weight[:D_FEAT], weight[D_FEAT:], bias)


# speedup vs baseline: 16.8446x; 1.0173x over previous
import functools
import math

import jax
import jax.numpy as jnp
from jax import lax
from jax.experimental import pallas as pl
from jax.experimental.pallas import tpu as pltpu
from jax.experimental.pallas import tpu_sc as plsc

N_NODES = 10000
N_EDGES = 320000
D_FEAT = 128

NC = 2
NS = 16
NW = NC * NS

EPW = N_EDGES // NW
K = 80
NCHUNK = EPW // K
NBUF = 3
NPAD = 10240
RPT = NPAD // NS
DUMMY = N_NODES
LANES = 16


def _sc_aggregate(nf_hbm, packed_hbm, acc_hbm, cnt_hbm,
                  packed_big, radj0, radj1, radj2, col0, col1, col2,
                  rows0, rows1, rows2, ones_v, acc_sh, cnt_sh,
                  gsem0, gsem1, gsem2, ssem0, ssem1, ssem2,
                  csem0, csem1, csem2):
    radj = [radj0, radj1, radj2]
    col = [col0, col1, col2]
    rows = [rows0, rows1, rows2]
    gsem = [gsem0, gsem1, gsem2]
    ssem = [ssem0, ssem1, ssem2]
    csem = [csem0, csem1, csem2]
    cid = lax.axis_index("c")
    sid = lax.axis_index("s")
    wid = sid * NC + cid

    zero16 = jnp.zeros((LANES,), jnp.float32)
    ebase = wid * EPW
    pltpu.sync_copy(packed_hbm.at[pl.ds(ebase, EPW)], packed_big)

    def _zrow(r, carry):
        for j in range(D_FEAT // LANES):
            rows[0][r, pl.ds(j * LANES, LANES)] = zero16
        return carry
    lax.fori_loop(0, K, _zrow, 0)

    one16 = jnp.ones((LANES,), jnp.float32)
    for i in range(K // LANES):
        ones_v[pl.ds(i * LANES, LANES)] = one16

    base_r = sid * RPT
    for k in range(RPT // K):
        pltpu.sync_copy(rows[0], acc_sh.at[pl.ds(base_r + k * K, K), :])
    for k in range(RPT // D_FEAT):
        pltpu.sync_copy(rows[0].at[0], cnt_sh.at[pl.ds(base_r + k * D_FEAT, D_FEAT)])

    plsc.subcore_barrier()

    def _unpack(base, i, radj_v, col_v):
        sl = pl.ds(base + i * LANES, LANES)
        p = packed_big[sl]
        cc = lax.bitwise_and(p, 16383)
        r = lax.shift_right_logical(p, 14)
        dst = pl.ds(i * LANES, LANES)
        col_v[dst] = cc
        radj_v[dst] = jnp.where(r == cc, DUMMY, r)

    def _prep(c, j):
        base = c * K
        for i in range(K // LANES):
            _unpack(base, i, radj[j], col[j])
        pltpu.async_copy(nf_hbm.at[col[j]], rows[j], gsem[j])

    def _wait_scat(j):
        pltpu.make_async_copy(rows[j], acc_sh.at[radj[j]], ssem[j]).wait()
        pltpu.make_async_copy(ones_v, cnt_sh.at[radj[j]], csem[j]).wait()

    def _scat(j):
        pltpu.make_async_copy(nf_hbm.at[col[j]], rows[j], gsem[j]).wait()
        pltpu.async_copy(rows[j], acc_sh.at[radj[j]], ssem[j], add=True)
        pltpu.async_copy(ones_v, cnt_sh.at[radj[j]], csem[j], add=True)

    def _body(c, j, first=False, last=False):
        jn = (j + 1) % NBUF
        if not first:
            _wait_scat(jn)
        if not last:
            _prep(c + 1, jn)
        _scat(j)

    _prep(0, 0)
    _body(0, 0, first=True)
    _body(1, 1, first=True)

    def _triple(g, carry):
        c = 3 * g + 2
        _body(c, 2)
        _body(c + 1, 0)
        _body(c + 2, 1)
        return carry
    lax.fori_loop(0, (NCHUNK - 5) // 3, _triple, 0)
    _body(NCHUNK - 3, 2)
    _body(NCHUNK - 2, 0)
    _body(NCHUNK - 1, 1, last=True)
    _wait_scat(0)
    _wait_scat(1)

    plsc.subcore_barrier()

    pltpu.sync_copy(acc_sh.at[pl.ds(base_r, RPT), :], acc_hbm.at[cid, pl.ds(base_r, RPT), :])
    pltpu.sync_copy(cnt_sh.at[pl.ds(base_r, RPT)], cnt_hbm.at[cid, pl.ds(base_r, RPT)])


@functools.cache
def _make_sc_call():
    return functools.partial(
        pl.kernel,
        mesh=plsc.VectorSubcoreMesh(core_axis_name="c", subcore_axis_name="s"),
        out_type=[
            jax.ShapeDtypeStruct((NC, NPAD, D_FEAT), jnp.float32),
            jax.ShapeDtypeStruct((NC, NPAD), jnp.float32),
        ],
        scratch_types=(
            [pltpu.VMEM((EPW,), jnp.int32)]
            + [pltpu.VMEM((K,), jnp.int32)] * NBUF
            + [pltpu.VMEM((K,), jnp.int32)] * NBUF
            + [pltpu.VMEM((K, D_FEAT), jnp.float32)] * NBUF
            + [pltpu.VMEM((K,), jnp.float32)]
            + [pltpu.VMEM_SHARED((NPAD, D_FEAT), jnp.float32)]
            + [pltpu.VMEM_SHARED((NPAD,), jnp.float32)]
            + [pltpu.SemaphoreType.DMA] * (3 * NBUF)
        ),
    )(_sc_aggregate)


BLK = 2048


def _tc_body(acc_ref, cnt_ref, nf_ref, wt_ref, wb_ref, b_ref, o_ref):
    cb = cnt_ref[0] + cnt_ref[1]
    inv = 1.0 / jnp.maximum(cb, 1.0)
    invb = jnp.broadcast_to(inv[:, :, None], (BLK // D_FEAT, D_FEAT, D_FEAT))
    s = (acc_ref[0] + acc_ref[1]).reshape(BLK // D_FEAT, D_FEAT, D_FEAT) * invb
    s = s.reshape(BLK, D_FEAT)
    y = (jnp.dot(s, wt_ref[...], preferred_element_type=jnp.float32)
         + jnp.dot(nf_ref[...], wb_ref[...], preferred_element_type=jnp.float32)
         + b_ref[...])
    nrm = jnp.sqrt(jnp.sum(y * y, axis=1, keepdims=True))
    o_ref[...] = y / jnp.maximum(nrm, 1e-12)


def _tc_finish(acc, cnt2d, nf, w_top, w_bot, bias):
    grid = NPAD // BLK
    return pl.pallas_call(
        _tc_body,
        grid=(grid,),
        in_specs=[
            pl.BlockSpec((NC, BLK, D_FEAT), lambda i: (0, i, 0)),
            pl.BlockSpec((NC, BLK // D_FEAT, D_FEAT), lambda i: (0, i, 0)),
            pl.BlockSpec((BLK, D_FEAT), lambda i: (i, 0)),
            pl.BlockSpec((D_FEAT, D_FEAT), lambda i: (0, 0)),
            pl.BlockSpec((D_FEAT, D_FEAT), lambda i: (0, 0)),
            pl.BlockSpec((1, D_FEAT), lambda i: (0, 0)),
        ],
        out_specs=pl.BlockSpec((BLK, D_FEAT), lambda i: (i, 0)),
        out_shape=jax.ShapeDtypeStruct((N_NODES, D_FEAT), jnp.float32),
    )(acc, cnt2d, nf, w_top, w_bot, bias)


def kernel(node_features, edge_index, weight, bias):
    row = edge_index[0]
    col = edge_index[1]
    packed = row * 16384 + col
    acc, cnt = _make_sc_call()(node_features, packed)
    cnt2d = cnt.reshape(NC, NPAD // D_FEAT, D_FEAT)
    return _tc_finish(acc, cnt2d, node_features, weight[:D_FEAT], weight[D_FEAT:], bias)
